# Initial kernel scaffold; baseline (speedup 1.0000x reference)
import jax
import jax.numpy as jnp
from jax.experimental import pallas as pl


def kernel(samples, alphas, Q):
    raise NotImplementedError("write your pallas kernel here")


# speedup vs baseline: 1.4846x; 1.4846x over previous
import math

import jax
import jax.numpy as jnp
from jax.experimental import pallas as pl

_DIM = 2048
_N_IN = 128
_SAMPLING_FACTOR = 4
_N_REP = _N_IN * _SAMPLING_FACTOR
_ENTROPY_PENALTY = 0.1
_BLK = 128


def _fused_kernel(alphas_ref, samples_ref, u_ref, q_ref, out_ref):
    i = pl.program_id(0)
    probs = (1.0 + jnp.cos(alphas_ref[...])) / 2.0
    s = samples_ref[...]
    u = u_ref[...]
    flips = (u < probs).astype(jnp.float32)
    flipped = flips * s + (1.0 - flips) * (1.0 - s)
    t = jnp.dot(flipped, q_ref[...], preferred_element_type=jnp.float32)
    part = jnp.sum(flipped * t)

    @pl.when(i == 0)
    def _init():
        out_ref[...] = jnp.zeros_like(out_ref)

    out_ref[...] += jnp.reshape(part, (1, 1))

    @pl.when(i == pl.num_programs(0) - 1)
    def _fin():
        p = probs + 1e-14
        ent = jnp.sum(p * jnp.log(1.0 / p))
        norm = _DIM * math.log(math.e) / math.e
        out_ref[...] = (out_ref[...] / _N_REP
                        + jnp.reshape(_ENTROPY_PENALTY * ent / norm, (1, 1)))


def kernel(samples, alphas, Q):
    fkey = jax.random.fold_in(jax.random.key(1), 123)
    u = jax.random.uniform(fkey, (_N_REP, _DIM), dtype=jnp.float32)
    out = pl.pallas_call(
        _fused_kernel,
        grid=(_N_REP // _BLK,),
        in_specs=[
            pl.BlockSpec((1, _DIM), lambda i: (0, 0)),
            pl.BlockSpec((_N_IN, _DIM), lambda i: (0, 0)),
            pl.BlockSpec((_BLK, _DIM), lambda i: (i, 0)),
            pl.BlockSpec((_DIM, _DIM), lambda i: (0, 0)),
        ],
        out_specs=pl.BlockSpec((1, 1), lambda i: (0, 0)),
        out_shape=jax.ShapeDtypeStruct((1, 1), jnp.float32),
    )(alphas.reshape(1, _DIM), samples, u, Q)
    return out.reshape(1)
